# baseline (speedup 1.0000x reference)
import functools

import jax
import jax.numpy as jnp
from jax import lax
from jax.experimental import pallas as pl
from jax.experimental.pallas import tpu as pltpu
from jax.experimental.pallas import tpu_sc as plsc

E = 16
TOP_K = 2
H = 1024
FF = 2048
FFC = 1024
NF = FF // FFC

L = 16
NC = 2
NS = 16


def _lane_max_splat(v, tmp_ref, lanes):
    for s in (1, 2, 4, 8):
        tmp_ref[...] = v
        v = jnp.maximum(v, plsc.load_gather(tmp_ref, [lanes ^ s]))
    return v


def _router_sc_body(x_hbm, gt_hbm, p_hbm, out_hbm, xv, gv, pv, av, tv):
    t = lax.axis_index("c") * NS + lax.axis_index("s")
    pltpu.sync_copy(x_hbm.at[t], xv)
    pltpu.sync_copy(gt_hbm, gv)
    for e in range(E):
        pltpu.sync_copy(p_hbm.at[e, t], pv.at[pl.ds(e * H, H)])
    lanes = lax.iota(jnp.int32, L)

    def body(j, acc):
        h = j * L
        xchunk = xv[pl.ds(h, L)]
        for k in range(L):
            acc = acc + xchunk[k] * gv[pl.ds((h + k) * E, L)]
        return acc

    scores = lax.fori_loop(0, H // L, body, jnp.zeros((L,), jnp.float32))
    m1 = _lane_max_splat(scores, tv, lanes)
    oh1 = lanes == plsc.all_reduce_ffs(scores == m1)
    masked = jnp.where(oh1, -jnp.inf, scores)
    m2 = _lane_max_splat(masked, tv, lanes)
    oh2 = lanes == plsc.all_reduce_ffs(masked == m2)
    e2 = jnp.exp(m2 - m1)
    denom = 1.0 + e2
    cw = jnp.where(oh1, 1.0 / denom, jnp.where(oh2, e2 / denom, 0.0))
    def comb_body(j, _):
        h = j * L
        acc = jnp.zeros((L,), jnp.float32)
        for e in range(E):
            acc = acc + cw[e] * pv[pl.ds(e * H + h, L)]
        av[pl.ds(h, L)] = acc
        return 0

    lax.fori_loop(0, H // L, comb_body, 0)
    pltpu.sync_copy(av, out_hbm.at[t])


def _router_combine_sc(x, gate_wT, partials):
    T = x.shape[0]
    return pl.kernel(
        _router_sc_body,
        out_type=jax.ShapeDtypeStruct((T, H), jnp.float32),
        mesh=plsc.VectorSubcoreMesh(core_axis_name="c", subcore_axis_name="s"),
        compiler_params=pltpu.CompilerParams(needs_layout_passes=False),
        scratch_types=[
            pltpu.VMEM((H,), jnp.float32),
            pltpu.VMEM((H * E,), jnp.float32),
            pltpu.VMEM((E * H,), jnp.float32),
            pltpu.VMEM((H,), jnp.float32),
            pltpu.VMEM((L,), jnp.float32),
        ],
    )(x, gate_wT, partials)


def _moe_body(x_ref, wg_ref, wu_ref, w2_ref, out_ref):
    f = pl.program_id(1)

    @pl.when(f == 0)
    def _init():
        out_ref[...] = jnp.zeros_like(out_ref)

    x = x_ref[...]
    g = jax.lax.dot_general(x, wg_ref[0], (((1,), (1,)), ((), ())),
                            preferred_element_type=jnp.float32)
    u = jax.lax.dot_general(x, wu_ref[0], (((1,), (1,)), ((), ())),
                            preferred_element_type=jnp.float32)
    act = (g * jax.nn.sigmoid(g)) * u
    ye = jax.lax.dot_general(act, w2_ref[0], (((1,), (1,)), ((), ())),
                             preferred_element_type=jnp.float32)
    out_ref[0] += ye


def _combine_body(p_ref, comb_ref, out_ref):
    comb = comb_ref[...]
    acc = jnp.zeros_like(out_ref)
    for e in range(E):
        oh_e = (jax.lax.broadcasted_iota(jnp.int32, (E, 1), 0) == e
                ).astype(jnp.float32)
        c = jax.lax.dot_general(comb, oh_e, (((1,), (0,)), ((), ())),
                                preferred_element_type=jnp.float32)
        acc = acc + p_ref[e] * c
    out_ref[...] = acc


@jax.jit
def kernel(hidden_states, gate_w, ws, w2s):
    b, s, h = hidden_states.shape
    x = hidden_states.reshape(-1, h)
    T = x.shape[0]

    grid = (E, NF)
    partials = pl.pallas_call(
        _moe_body,
        grid=grid,
        in_specs=[
            pl.BlockSpec((T, H), lambda e, f: (0, 0)),
            pl.BlockSpec((1, FFC, H), lambda e, f: (e, f, 0)),
            pl.BlockSpec((1, FFC, H), lambda e, f: (e, NF + f, 0)),
            pl.BlockSpec((1, H, FFC), lambda e, f: (e, 0, f)),
        ],
        out_specs=pl.BlockSpec((1, T, H), lambda e, f: (e, 0, 0)),
        out_shape=jax.ShapeDtypeStruct((E, T, H), jnp.float32),
        compiler_params=pltpu.CompilerParams(
            dimension_semantics=("arbitrary", "arbitrary"),
        ),
        cost_estimate=pl.CostEstimate(
            flops=2 * 32 * E * (2 * FF * H + H * FF), transcendentals=32 * E * FF,
            bytes_accessed=E * (3 * FF * H) * 4),
    )(x, ws, ws, w2s)

    out = _router_combine_sc(x, gate_w.T.reshape(-1), partials)
    return out.reshape(b, s, h)


# speedup vs baseline: 1.0867x; 1.0867x over previous
import functools

import jax
import jax.numpy as jnp
from jax import lax
from jax.experimental import pallas as pl
from jax.experimental.pallas import tpu as pltpu
from jax.experimental.pallas import tpu_sc as plsc

E = 16
TOP_K = 2
H = 1024
FF = 2048
FFC = 1024
NF = FF // FFC

L = 16
NC = 2
NS = 16


def _lane_max_splat(v, tmp_ref, lanes):
    for s in (1, 2, 4, 8):
        tmp_ref[...] = v
        v = jnp.maximum(v, plsc.load_gather(tmp_ref, [lanes ^ s]))
    return v


def _router_sc_body(x_hbm, gt_hbm, comb_hbm, xv, gv, cv, tv, sem1, sem2):
    t = lax.axis_index("c") * NS + lax.axis_index("s")
    cp1 = pltpu.async_copy(x_hbm.at[t], xv, sem1)
    cp2 = pltpu.async_copy(gt_hbm, gv, sem2)
    cp1.wait()
    cp2.wait()
    lanes = lax.iota(jnp.int32, L)

    def body(j, acc):
        h = j * L
        xchunk = xv[pl.ds(h, L)]
        for k in range(L):
            acc = acc + xchunk[k] * gv[pl.ds((h + k) * E, L)]
        return acc

    scores = lax.fori_loop(0, H // L, body, jnp.zeros((L,), jnp.float32))
    m1 = _lane_max_splat(scores, tv, lanes)
    oh1 = lanes == plsc.all_reduce_ffs(scores == m1)
    masked = jnp.where(oh1, -jnp.inf, scores)
    m2 = _lane_max_splat(masked, tv, lanes)
    oh2 = lanes == plsc.all_reduce_ffs(masked == m2)
    e2 = jnp.exp(m2 - m1)
    denom = 1.0 + e2
    cv[...] = jnp.where(oh1, 1.0 / denom, jnp.where(oh2, e2 / denom, 0.0))
    pltpu.sync_copy(cv, comb_hbm.at[t])


def _router_sc(x, gate_wT):
    T = x.shape[0]
    return pl.kernel(
        _router_sc_body,
        out_type=jax.ShapeDtypeStruct((T, E), jnp.float32),
        mesh=plsc.VectorSubcoreMesh(core_axis_name="c", subcore_axis_name="s"),
        compiler_params=pltpu.CompilerParams(needs_layout_passes=False),
        cost_estimate=pl.CostEstimate(
            flops=2 * 32 * E * H, transcendentals=32,
            bytes_accessed=(32 * H + H * E + 32 * E) * 4),
        scratch_types=[
            pltpu.VMEM((H,), jnp.float32),
            pltpu.VMEM((H * E,), jnp.float32),
            pltpu.VMEM((L,), jnp.float32),
            pltpu.VMEM((L,), jnp.float32),
            pltpu.SemaphoreType.DMA,
            pltpu.SemaphoreType.DMA,
        ],
    )(x, gate_wT)


def _moe_body(x_ref, wg_ref, wu_ref, w2_ref, out_ref):
    f = pl.program_id(1)

    @pl.when(f == 0)
    def _init():
        out_ref[...] = jnp.zeros_like(out_ref)

    x = x_ref[...]
    g = jax.lax.dot_general(x, wg_ref[0], (((1,), (1,)), ((), ())),
                            preferred_element_type=jnp.float32)
    u = jax.lax.dot_general(x, wu_ref[0], (((1,), (1,)), ((), ())),
                            preferred_element_type=jnp.float32)
    act = (g * jax.nn.sigmoid(g)) * u
    ye = jax.lax.dot_general(act, w2_ref[0], (((1,), (1,)), ((), ())),
                             preferred_element_type=jnp.float32)
    out_ref[0] += ye


def _combine_body(p_ref, comb_ref, out_ref):
    comb = comb_ref[...]
    acc = jnp.zeros_like(out_ref)
    for e in range(E):
        oh_e = (jax.lax.broadcasted_iota(jnp.int32, (E, 1), 0) == e
                ).astype(jnp.float32)
        c = jax.lax.dot_general(comb, oh_e, (((1,), (0,)), ((), ())),
                                preferred_element_type=jnp.float32)
        acc = acc + p_ref[e] * c
    out_ref[...] = acc


@jax.jit
def kernel(hidden_states, gate_w, ws, w2s):
    b, s, h = hidden_states.shape
    x = hidden_states.reshape(-1, h)
    T = x.shape[0]

    comb = _router_sc(x, gate_w.T.reshape(-1))

    grid = (E, NF)
    partials = pl.pallas_call(
        _moe_body,
        grid=grid,
        in_specs=[
            pl.BlockSpec((T, H), lambda e, f: (0, 0)),
            pl.BlockSpec((1, FFC, H), lambda e, f: (e, f, 0)),
            pl.BlockSpec((1, FFC, H), lambda e, f: (e, NF + f, 0)),
            pl.BlockSpec((1, H, FFC), lambda e, f: (e, 0, f)),
        ],
        out_specs=pl.BlockSpec((1, T, H), lambda e, f: (e, 0, 0)),
        out_shape=jax.ShapeDtypeStruct((E, T, H), jnp.float32),
        compiler_params=pltpu.CompilerParams(
            dimension_semantics=("arbitrary", "arbitrary"),
        ),
        cost_estimate=pl.CostEstimate(
            flops=2 * 32 * E * (2 * FF * H + H * FF), transcendentals=32 * E * FF,
            bytes_accessed=E * (3 * FF * H) * 4),
    )(x, ws, ws, w2s)

    out = pl.pallas_call(
        _combine_body,
        out_shape=jax.ShapeDtypeStruct((T, H), jnp.float32),
    )(partials, comb)
    return out.reshape(b, s, h)
